# Initial kernel scaffold; baseline (speedup 1.0000x reference)
import jax
import jax.numpy as jnp
from jax.experimental import pallas as pl


def kernel(h, W):
    raise NotImplementedError("write your pallas kernel here")


# speedup vs baseline: 1.1333x; 1.1333x over previous
import functools

import jax
import jax.numpy as jnp
from jax.experimental import pallas as pl

_E = 64
_K = 8
_NEG = -3.0e38


def _gate_block(h_ref, w_ref, ids_ref, wts_ref):
    h = h_ref[...]
    w = w_ref[...]
    logits = jax.lax.dot_general(
        h, w, (((1,), (1,)), ((), ())),
        preferred_element_type=jnp.float32)
    b = logits.shape[0]
    lane = jax.lax.broadcasted_iota(jnp.int32, (b, _E), 1)
    work = logits
    ids = []
    vals = []
    for _ in range(_K):
        m = jnp.max(work, axis=1, keepdims=True)
        idx = jnp.min(jnp.where(work == m, lane, _E),
                      axis=1, keepdims=True)
        ids.append(idx)
        vals.append(m)
        work = jnp.where(lane == idx, _NEG, work)
    v = jnp.concatenate(vals, axis=1)
    e = jnp.exp(v - v[:, :1])
    wts = e / jnp.sum(e, axis=1, keepdims=True)
    ids_ref[...] = jnp.concatenate(ids, axis=1)
    wts_ref[...] = wts


@functools.partial(jax.jit, static_argnames=())
def kernel(h, W):
    b, s, d = h.shape
    n = b * s
    hf = h.reshape(n, d)
    blk = 1024
    grid = n // blk
    ids, wts = pl.pallas_call(
        _gate_block,
        grid=(grid,),
        in_specs=[
            pl.BlockSpec((blk, d), lambda i: (i, 0)),
            pl.BlockSpec((_E, d), lambda i: (0, 0)),
        ],
        out_specs=[
            pl.BlockSpec((blk, _K), lambda i: (i, 0)),
            pl.BlockSpec((blk, _K), lambda i: (i, 0)),
        ],
        out_shape=[
            jax.ShapeDtypeStruct((n, _K), jnp.int32),
            jax.ShapeDtypeStruct((n, _K), jnp.float32),
        ],
    )(hf, W)
    return ids, wts, jnp.float32(0.0)


# speedup vs baseline: 1.9554x; 1.7253x over previous
import functools

import jax
import jax.numpy as jnp
from jax.experimental import pallas as pl

_E = 64
_K = 8
_NEG = -3.0e38


def _gate_block(h_ref, w_ref, ids_ref, wts_ref):
    h = h_ref[...]
    w = w_ref[...]
    logits = jax.lax.dot_general(
        h, w, (((1,), (1,)), ((), ())),
        preferred_element_type=jnp.float32)
    b = logits.shape[0]
    lane = jax.lax.broadcasted_iota(jnp.int32, (b, _E), 1)
    raw = jax.lax.bitcast_convert_type(logits, jnp.int32)
    low6 = jnp.where(raw < 0, lane, (_E - 1) - lane)
    key = jax.lax.bitcast_convert_type((raw & ~(_E - 1)) | low6, jnp.float32)
    ids = []
    vals = []
    for _ in range(_K):
        m = jnp.max(key, axis=1, keepdims=True)
        mi = jax.lax.bitcast_convert_type(m, jnp.int32)
        ids.append(jnp.where(mi < 0, mi & (_E - 1),
                             (_E - 1) - (mi & (_E - 1))))
        vals.append(jax.lax.bitcast_convert_type(mi & ~(_E - 1),
                                                 jnp.float32))
        key = jnp.where(key == m, _NEG, key)
    v = jnp.concatenate(vals, axis=1)
    e = jnp.exp(v - v[:, :1])
    wts = e / jnp.sum(e, axis=1, keepdims=True)
    ids_ref[...] = jnp.concatenate(ids, axis=1)
    wts_ref[...] = wts


@functools.partial(jax.jit, static_argnames=())
def kernel(h, W):
    b, s, d = h.shape
    n = b * s
    hf = h.reshape(n, d)
    blk = 1024
    grid = n // blk
    ids, wts = pl.pallas_call(
        _gate_block,
        grid=(grid,),
        in_specs=[
            pl.BlockSpec((blk, d), lambda i: (i, 0)),
            pl.BlockSpec((_E, d), lambda i: (0, 0)),
        ],
        out_specs=[
            pl.BlockSpec((blk, _K), lambda i: (i, 0)),
            pl.BlockSpec((blk, _K), lambda i: (i, 0)),
        ],
        out_shape=[
            jax.ShapeDtypeStruct((n, _K), jnp.int32),
            jax.ShapeDtypeStruct((n, _K), jnp.float32),
        ],
    )(hf, W)
    return ids, wts, jnp.float32(0.0)


# speedup vs baseline: 2.3588x; 1.2063x over previous
import functools

import jax
import jax.numpy as jnp
from jax.experimental import pallas as pl

_E = 64
_K = 8
_NEG = -3.0e38


def _gate_block(h_ref, w_ref, ids_ref, wts_ref):
    h = h_ref[...]
    w = w_ref[...]
    logits = jax.lax.dot_general(
        w, h, (((1,), (1,)), ((), ())),
        preferred_element_type=jnp.float32)
    b = logits.shape[1]
    sub = jax.lax.broadcasted_iota(jnp.int32, (_E, b), 0)
    raw = jax.lax.bitcast_convert_type(logits, jnp.int32)
    low6 = jnp.where(raw < 0, sub, (_E - 1) - sub)
    key = jax.lax.bitcast_convert_type((raw & ~(_E - 1)) | low6, jnp.float32)
    ms = []
    for _ in range(_K):
        m = jnp.max(key, axis=0, keepdims=True)
        ms.append(m)
        key = jnp.where(key == m, _NEG, key)
    packed = jnp.concatenate(ms, axis=0)
    mi = jax.lax.bitcast_convert_type(packed, jnp.int32)
    low = mi & (_E - 1)
    ids_t = jnp.where(mi < 0, low, (_E - 1) - low)
    vals_t = jax.lax.bitcast_convert_type(mi & ~(_E - 1), jnp.float32)
    e = jnp.exp(vals_t - vals_t[:1, :])
    wts_t = e / jnp.sum(e, axis=0, keepdims=True)
    ids_ref[...] = ids_t.T
    wts_ref[...] = wts_t.T


@functools.partial(jax.jit, static_argnames=())
def kernel(h, W):
    b, s, d = h.shape
    n = b * s
    hf = h.reshape(n, d)
    blk = 1024
    grid = n // blk
    ids, wts = pl.pallas_call(
        _gate_block,
        grid=(grid,),
        in_specs=[
            pl.BlockSpec((blk, d), lambda i: (i, 0)),
            pl.BlockSpec((_E, d), lambda i: (0, 0)),
        ],
        out_specs=[
            pl.BlockSpec((blk, _K), lambda i: (i, 0)),
            pl.BlockSpec((blk, _K), lambda i: (i, 0)),
        ],
        out_shape=[
            jax.ShapeDtypeStruct((n, _K), jnp.int32),
            jax.ShapeDtypeStruct((n, _K), jnp.float32),
        ],
    )(hf, W)
    return ids, wts, jnp.float32(0.0)


# speedup vs baseline: 2.7009x; 1.1450x over previous
import functools

import jax
import jax.numpy as jnp
from jax.experimental import pallas as pl

_E = 64
_K = 8
_NEG = -3.0e38


def _gate_block(h_ref, w_ref, ids_ref, wts_ref):
    h = h_ref[...]
    w = w_ref[...]
    logits = jax.lax.dot_general(
        w, h, (((1,), (1,)), ((), ())),
        preferred_element_type=jnp.float32)
    b = logits.shape[1]
    sub = jax.lax.broadcasted_iota(jnp.int32, (_E, b), 0)
    raw = jax.lax.bitcast_convert_type(logits, jnp.int32)
    low6 = jnp.where(raw < 0, sub, (_E - 1) - sub)
    key = jax.lax.bitcast_convert_type((raw & ~(_E - 1)) | low6, jnp.float32)
    ms = []
    for _ in range(_K):
        m = jnp.max(key, axis=0, keepdims=True)
        ms.append(m)
        key = jnp.where(key == m, _NEG, key)
    packed = jnp.concatenate(ms, axis=0)
    mi = jax.lax.bitcast_convert_type(packed, jnp.int32)
    low = mi & (_E - 1)
    ids_t = jnp.where(mi < 0, low, (_E - 1) - low)
    vals_t = jax.lax.bitcast_convert_type(mi & ~(_E - 1), jnp.float32)
    e = jnp.exp(vals_t - vals_t[:1, :])
    wts_t = e / jnp.sum(e, axis=0, keepdims=True)
    ids_ref[...] = ids_t.T
    wts_ref[...] = wts_t.T


@functools.partial(jax.jit, static_argnames=())
def kernel(h, W):
    b, s, d = h.shape
    n = b * s
    hf = h.reshape(n, d)
    blk = 2048
    grid = n // blk
    ids, wts = pl.pallas_call(
        _gate_block,
        grid=(grid,),
        in_specs=[
            pl.BlockSpec((blk, d), lambda i: (i, 0)),
            pl.BlockSpec((_E, d), lambda i: (0, 0)),
        ],
        out_specs=[
            pl.BlockSpec((blk, _K), lambda i: (i, 0)),
            pl.BlockSpec((blk, _K), lambda i: (i, 0)),
        ],
        out_shape=[
            jax.ShapeDtypeStruct((n, _K), jnp.int32),
            jax.ShapeDtypeStruct((n, _K), jnp.float32),
        ],
    )(hf, W)
    return ids, wts, jnp.float32(0.0)


# speedup vs baseline: 2.8431x; 1.0526x over previous
import functools

import jax
import jax.numpy as jnp
from jax.experimental import pallas as pl

_E = 64
_K = 8
_NEG = -3.0e38


def _gate_block(h_ref, w_ref, ids_ref, wts_ref):
    h = h_ref[...]
    w = w_ref[...]
    logits = jax.lax.dot_general(
        w, h, (((1,), (1,)), ((), ())),
        preferred_element_type=jnp.float32)
    b = logits.shape[1]
    sub = jax.lax.broadcasted_iota(jnp.int32, (_E, b), 0)
    raw = jax.lax.bitcast_convert_type(logits, jnp.int32)
    low6 = jnp.where(raw < 0, sub, (_E - 1) - sub)
    key = jax.lax.bitcast_convert_type((raw & ~(_E - 1)) | low6, jnp.float32)
    ms = []
    for _ in range(_K):
        m = jnp.max(key, axis=0, keepdims=True)
        ms.append(m)
        key = jnp.where(key == m, _NEG, key)
    packed = jnp.concatenate(ms, axis=0)
    mi = jax.lax.bitcast_convert_type(packed, jnp.int32)
    low = mi & (_E - 1)
    ids_t = jnp.where(mi < 0, low, (_E - 1) - low)
    vals_t = jax.lax.bitcast_convert_type(mi & ~(_E - 1), jnp.float32)
    e = jnp.exp(vals_t - vals_t[:1, :])
    wts_t = e / jnp.sum(e, axis=0, keepdims=True)
    ids_ref[...] = ids_t.T
    wts_ref[...] = wts_t.T


@functools.partial(jax.jit, static_argnames=())
def kernel(h, W):
    b, s, d = h.shape
    n = b * s
    hf = h.reshape(n, d)
    blk = 4096
    grid = n // blk
    ids, wts = pl.pallas_call(
        _gate_block,
        grid=(grid,),
        in_specs=[
            pl.BlockSpec((blk, d), lambda i: (i, 0)),
            pl.BlockSpec((_E, d), lambda i: (0, 0)),
        ],
        out_specs=[
            pl.BlockSpec((blk, _K), lambda i: (i, 0)),
            pl.BlockSpec((blk, _K), lambda i: (i, 0)),
        ],
        out_shape=[
            jax.ShapeDtypeStruct((n, _K), jnp.int32),
            jax.ShapeDtypeStruct((n, _K), jnp.float32),
        ],
    )(hf, W)
    return ids, wts, jnp.float32(0.0)
